# baseline (speedup 1.0000x reference)
import jax
import jax.numpy as jnp
from jax import lax
from jax.experimental import pallas as pl

_B, _H, _S, _DH = 8, 16, 4096, 64
_DM = _H * _DH
_NR = 128
_R, _C = 32, 128
_SP = _S // 2


def _attn_body(q_ref, k_ref, v_ref, r1_ref, r2_ref, out_ref):
    kk = k_ref[...]
    qv = q_ref[0]
    r1 = r1_ref[...]
    r2 = r2_ref[...]

    zmat = jnp.zeros((_DH, _DH), jnp.float32)
    r1e = jnp.concatenate(
        [jnp.concatenate([r1, zmat], 1), jnp.concatenate([zmat, r1], 1)], 0)
    r2e = jnp.concatenate(
        [jnp.concatenate([r2, zmat], 1), jnp.concatenate([zmat, r2], 1)], 0)

    kh = jnp.maximum(jnp.dot(kk, r1e), 0.0) @ r2e
    qh = jnp.maximum(jnp.dot(qv, r1), 0.0) @ r2
    sgn_k = jnp.where(kh > 0, 1.0, -1.0)
    sgn_q = jnp.where(qh > 0, 1.0, -1.0)

    zrow = jnp.zeros((1, _DH), jnp.float32)
    sq_e = jnp.concatenate([sgn_q, zrow], 1)
    sq_o = jnp.concatenate([zrow, sgn_q], 1)
    qv_e = jnp.concatenate([qv, zrow], 1)
    qv_o = jnp.concatenate([zrow, qv], 1)

    dn = (((1,), (1,)), ((), ()))
    m = jnp.concatenate(
        [lax.dot_general(sq_e, sgn_k, dn).reshape(_R // 2, _C),
         lax.dot_general(sq_o, sgn_k, dn).reshape(_R // 2, _C)], 0)
    qk = jnp.concatenate(
        [lax.dot_general(qv_e, kk, dn).reshape(_R // 2, _C),
         lax.dot_general(qv_o, kk, dn).reshape(_R // 2, _C)], 0) * 0.125

    half = lax.broadcasted_iota(jnp.int32, (_R, _C), 0) >= (_R // 2)
    pi = (lax.broadcasted_iota(jnp.int32, (_R, _C), 0) % (_R // 2)) * _C \
        + lax.broadcasted_iota(jnp.int32, (_R, _C), 1)
    sidx = (2 * pi + half.astype(jnp.int32)).astype(jnp.float32)

    one = jnp.ones((1, 1), jnp.float32)

    def _t_step(_, carry):
        lo, hi = carry
        mid = jnp.floor((lo + hi + one) * 0.5)
        cnt = jnp.sum((m >= mid).astype(jnp.float32), keepdims=True)
        ok = cnt >= _NR
        return (jnp.where(ok, mid, lo), jnp.where(ok, hi, mid - one))

    t, _ = lax.fori_loop(0, 8, _t_step, (-64.0 * one, 64.0 * one))

    gt = m > t
    n_gt = jnp.sum(gt.astype(jnp.float32), keepdims=True)
    k_rem = _NR - n_gt

    eq = m == t

    def _c_step(_, carry):
        lo, hi = carry
        mid = jnp.floor((lo + hi) * 0.5)
        cnt = jnp.sum(jnp.where(eq & (sidx < mid), 1.0, 0.0), keepdims=True)
        ok = cnt >= k_rem
        return (jnp.where(ok, lo, mid + one), jnp.where(ok, mid, hi))

    _, cstar = lax.fori_loop(0, 13, _c_step,
                             (0.0 * one, jnp.float32(_S) * one))

    mask = gt | (eq & (sidx < cstar))

    sm = jnp.where(mask, qk, -jnp.inf)
    p = jnp.exp(sm - jnp.max(sm, keepdims=True))
    w = p / jnp.sum(p, keepdims=True)
    w_e = w[:_R // 2].reshape(1, _SP)
    w_o = w[_R // 2:].reshape(1, _SP)
    vv = v_ref[...]
    ae = jnp.dot(w_e, vv)
    ao = jnp.dot(w_o, vv)
    out_ref[0] = ae[:, :_DH] + ao[:, _DH:]


def _proj_body(a_ref, w_ref, o_ref):
    o_ref[...] = lax.dot_general(a_ref[...], w_ref[...],
                                 (((1,), (1,)), ((), ())))


def kernel(q, k, v, rot_mat1, rot_mat2, W_o):
    BH = _B * _H
    q3 = q.reshape(BH, 1, _DH)
    k2 = k.reshape(BH * _SP, 2 * _DH)
    v2 = v.reshape(BH * _SP, 2 * _DH)

    attn = pl.pallas_call(
        _attn_body,
        grid=(BH,),
        in_specs=[
            pl.BlockSpec((1, 1, _DH), lambda i: (i, 0, 0)),
            pl.BlockSpec((_SP, 2 * _DH), lambda i: (i, 0)),
            pl.BlockSpec((_SP, 2 * _DH), lambda i: (i, 0)),
            pl.BlockSpec((_DH, _DH), lambda i: (0, 0)),
            pl.BlockSpec((_DH, _DH), lambda i: (0, 0)),
        ],
        out_specs=pl.BlockSpec((1, 1, _DH), lambda i: (i, 0, 0)),
        out_shape=jax.ShapeDtypeStruct((BH, 1, _DH), jnp.float32),
    )(q3, k2, v2, rot_mat1, rot_mat2)

    attn2 = attn.reshape(_B, _DM)
    out = pl.pallas_call(
        _proj_body,
        out_shape=jax.ShapeDtypeStruct((_B, _DM), jnp.float32),
    )(attn2, W_o)
    return out.reshape(_B, 1, _DM)


# speedup vs baseline: 1.4492x; 1.4492x over previous
import jax
import jax.numpy as jnp
from jax import lax
from jax.experimental import pallas as pl

_B, _H, _S, _DH = 8, 16, 4096, 64
_DM = _H * _DH
_NR = 128
_R, _C = 32, 128


_G = 4


def _attn_body(q_ref, k_ref, v_ref, r1_ref, r2_ref, out_ref):
    i = pl.program_id(0)
    r1 = r1_ref[...]
    r2 = r2_ref[...]
    for g in range(_G):
        _one_head(q_ref, k_ref, v_ref, r1, r2, out_ref, i * _G + g, g)


def _one_head(q_ref, k_ref, v_ref, r1, r2, out_ref, bh, g):
    kk = k_ref[pl.ds(g * _S, _S), :]
    qv = q_ref[bh]

    kh = jnp.maximum(jnp.dot(kk, r1), 0.0) @ r2
    qh = jnp.maximum(jnp.dot(qv, r1), 0.0) @ r2
    sgn_k = jnp.where(kh > 0, 1.0, -1.0)
    sgn_q = jnp.where(qh > 0, 1.0, -1.0)
    dn = (((1,), (1,)), ((), ()))
    m = lax.dot_general(sgn_q, sgn_k, dn).reshape(_R, _C)
    qk = (lax.dot_general(qv, kk, dn) * 0.125).reshape(_R, _C)

    one = jnp.ones((1, 1), jnp.float32)

    def _t_step(_, carry):
        lo, hi = carry
        mid = jnp.floor((lo + hi + one) * 0.5)
        cnt = jnp.sum((m >= mid).astype(jnp.float32), keepdims=True)
        ok = cnt >= _NR
        return (jnp.where(ok, mid, lo), jnp.where(ok, hi, mid - one))

    t, _ = lax.fori_loop(0, 8, _t_step, (-64.0 * one, 64.0 * one))

    gt = m > t
    n_gt = jnp.sum(gt.astype(jnp.float32), keepdims=True)
    k_rem = _NR - n_gt

    eq = m == t
    sidx = (lax.broadcasted_iota(jnp.int32, (_R, _C), 0) * _C
            + lax.broadcasted_iota(jnp.int32, (_R, _C), 1)
            ).astype(jnp.float32)

    def _c_step(_, carry):
        lo, hi = carry
        mid = jnp.floor((lo + hi) * 0.5)
        cnt = jnp.sum(jnp.where(eq & (sidx < mid), 1.0, 0.0), keepdims=True)
        ok = cnt >= k_rem
        return (jnp.where(ok, lo, mid + one), jnp.where(ok, mid, hi))

    _, cstar = lax.fori_loop(0, 13, _c_step,
                             (0.0 * one, jnp.float32(_S) * one))

    mask = gt | (eq & (sidx < cstar))

    sm = jnp.where(mask, qk, -jnp.inf)
    p = jnp.exp(sm - jnp.max(sm, keepdims=True))
    w = (p / jnp.sum(p, keepdims=True)).reshape(1, _S)
    vv = v_ref[pl.ds(g * _S, _S), :]
    out_ref[bh] = jnp.dot(w, vv)


def _proj_body(a_ref, w_ref, o_ref):
    o_ref[...] = lax.dot_general(a_ref[...], w_ref[...],
                                 (((1,), (1,)), ((), ())))


def kernel(q, k, v, rot_mat1, rot_mat2, W_o):
    BH = _B * _H
    q3 = q.reshape(BH, 1, _DH)
    k2 = k.reshape(BH * _S, _DH)
    v2 = v.reshape(BH * _S, _DH)

    attn = pl.pallas_call(
        _attn_body,
        grid=(BH // _G,),
        in_specs=[
            pl.BlockSpec((BH, 1, _DH), lambda i: (0, 0, 0)),
            pl.BlockSpec((_G * _S, _DH), lambda i: (i, 0)),
            pl.BlockSpec((_G * _S, _DH), lambda i: (i, 0)),
            pl.BlockSpec((_DH, _DH), lambda i: (0, 0)),
            pl.BlockSpec((_DH, _DH), lambda i: (0, 0)),
        ],
        out_specs=pl.BlockSpec((BH, 1, _DH), lambda i: (0, 0, 0)),
        out_shape=jax.ShapeDtypeStruct((BH, 1, _DH), jnp.float32),
    )(q3, k2, v2, rot_mat1, rot_mat2)

    attn2 = attn.reshape(_B, _DM)
    out = pl.pallas_call(
        _proj_body,
        out_shape=jax.ShapeDtypeStruct((_B, _DM), jnp.float32),
    )(attn2, W_o)
    return out.reshape(_B, 1, _DM)
